# baseline (speedup 1.0000x reference)
import functools

import jax
import jax.numpy as jnp
from jax import lax
from jax.experimental import pallas as pl
from jax.experimental.pallas import tpu as pltpu
from jax.experimental.pallas import tpu_sc as plsc

B = 16384
A = 200
NC = 2
NS = 16
L = 16
NW = NC * NS

MPW = 128
SC_B = MPW * NW
CGROUPS = MPW // L
NACC = 8

TC_B = B - SC_B
TC_BLK = 2048


def _take16(table_vec, idx):
    return lax.gather(
        table_vec,
        idx[:, None],
        dimension_numbers=lax.GatherDimensionNumbers(
            offset_dims=(),
            collapsed_slice_dims=(0,),
            start_index_map=(0,),
        ),
        slice_sizes=(1,),
        mode=lax.GatherScatterMode.PROMISE_IN_BOUNDS,
    )


def _sc_body(energies_hbm, species_t_hbm, table_hbm, out_hbm,
             species_v, energies_v, out_v, table_v):
    wid = lax.axis_index("s") * NC + lax.axis_index("c")
    base = wid * MPW

    pltpu.sync_copy(species_t_hbm.at[:, pl.ds(base, MPW)], species_v)
    pltpu.sync_copy(table_hbm, table_v)
    pltpu.sync_copy(energies_hbm.at[pl.ds(base, MPW)], energies_v)

    table_vec = table_v[...]
    iota = lax.iota(jnp.int32, L)
    zeros_f = jnp.zeros((L,), jnp.float32)
    zeros_i = jnp.zeros((L,), jnp.int32)

    def group_fn(g, _):
        mols = iota + g * L

        @plsc.parallel_loop(0, A, carry=(zeros_i, (zeros_f,) * NACC),
                            unroll=8)
        def loop(_, carry):
            atom, accs = carry
            s = plsc.load_gather(species_v, [atom, mols])
            v = _take16(table_vec, s)
            return atom + 1, accs[1:] + (accs[0] + v,)

        _, accs = loop
        acc = ((accs[0] + accs[1]) + (accs[2] + accs[3])) + (
            (accs[4] + accs[5]) + (accs[6] + accs[7]))
        off = g * L
        e = energies_v[pl.ds(off, L)]
        out_v[pl.ds(off, L)] = e - acc
        return 0

    lax.fori_loop(0, CGROUPS, group_fn, 0)
    pltpu.sync_copy(out_v, out_hbm.at[pl.ds(base, MPW)])


def _sc_part(energies, species_t, table16):
    mesh = plsc.VectorSubcoreMesh(
        core_axis_name="c", subcore_axis_name="s",
        num_cores=NC, num_subcores=NS,
    )
    f = functools.partial(
        pl.kernel,
        mesh=mesh,
        compiler_params=pltpu.CompilerParams(needs_layout_passes=False),
        out_type=jax.ShapeDtypeStruct((SC_B,), jnp.float32),
        scratch_types=[
            pltpu.VMEM((A, MPW), jnp.int32),
            pltpu.VMEM((MPW,), jnp.float32),
            pltpu.VMEM((MPW,), jnp.float32),
            pltpu.VMEM((L,), jnp.float32),
        ],
    )(_sc_body)
    return f(energies, species_t, table16)


def _tc_body(table_ref, energies_ref, species_t_ref, out_ref):
    t0 = table_ref[0]
    d1 = table_ref[1] - t0
    d2 = table_ref[2] - t0
    d3 = table_ref[3] - t0
    s = species_t_ref[...]
    val = jnp.where(s == 1, d1, 0.0)
    val = val + jnp.where(s == 2, d2, 0.0)
    val = val + jnp.where(s == 3, d3, 0.0)
    sae = jnp.sum(val, axis=0) + jnp.float32(A) * t0
    out_ref[...] = energies_ref[...] - sae


def _tc_part(energies, species_t, table4):
    grid = (TC_B // TC_BLK,)
    off = SC_B // TC_BLK
    return pl.pallas_call(
        _tc_body,
        grid_spec=pltpu.PrefetchScalarGridSpec(
            num_scalar_prefetch=1,
            grid=grid,
            in_specs=[
                pl.BlockSpec((TC_BLK,), lambda i, t: (i + off,)),
                pl.BlockSpec((A, TC_BLK), lambda i, t: (0, i + off)),
            ],
            out_specs=pl.BlockSpec((TC_BLK,), lambda i, t: (i,)),
        ),
        out_shape=jax.ShapeDtypeStruct((TC_B,), jnp.float32),
    )(table4, energies, species_t)


@jax.jit
def _sae_kernel(energies, species, table16, table4):
    species_t = lax.transpose(species, (1, 0))
    sc_out = _sc_part(energies, species_t, table16)
    tc_out = _tc_part(energies, species_t, table4)
    return jnp.concatenate([sc_out, tc_out])


def kernel(energies, species, self_energies):
    table4 = self_energies.astype(jnp.float32)
    table16 = jnp.zeros((L,), jnp.float32).at[:4].set(table4)
    return _sae_kernel(energies, species.astype(jnp.int32), table16, table4)


# speedup vs baseline: 1.0101x; 1.0101x over previous
import functools

import jax
import jax.numpy as jnp
from jax import lax
from jax.experimental import pallas as pl
from jax.experimental.pallas import tpu as pltpu
from jax.experimental.pallas import tpu_sc as plsc

B = 16384
A = 200
NC = 2
NS = 16
L = 16
NW = NC * NS

MPW = 128
SC_B = MPW * NW
CGROUPS = MPW // L
NACC = 8

TC_B = B - SC_B
TC_BLK = 2048


def _take16(table_vec, idx):
    return lax.gather(
        table_vec,
        idx[:, None],
        dimension_numbers=lax.GatherDimensionNumbers(
            offset_dims=(),
            collapsed_slice_dims=(0,),
            start_index_map=(0,),
        ),
        slice_sizes=(1,),
        mode=lax.GatherScatterMode.PROMISE_IN_BOUNDS,
    )


def _sc_body(energies_hbm, species_t_hbm, table_hbm, out_hbm,
             species_v, energies_v, out_v, table_v):
    wid = lax.axis_index("s") * NC + lax.axis_index("c")
    base = wid * MPW

    pltpu.sync_copy(species_t_hbm.at[:, pl.ds(base, MPW)], species_v)
    pltpu.sync_copy(table_hbm, table_v.at[pl.ds(0, 4)])
    pltpu.sync_copy(energies_hbm.at[pl.ds(base, MPW)], energies_v)

    table_vec = table_v[...]
    iota = lax.iota(jnp.int32, L)
    zeros_f = jnp.zeros((L,), jnp.float32)
    zeros_i = jnp.zeros((L,), jnp.int32)

    def group_fn(g, _):
        mols = iota + g * L

        @plsc.parallel_loop(0, A, carry=(zeros_i, (zeros_f,) * NACC),
                            unroll=8)
        def loop(_, carry):
            atom, accs = carry
            s = plsc.load_gather(species_v, [atom, mols])
            v = _take16(table_vec, s)
            return atom + 1, accs[1:] + (accs[0] + v,)

        _, accs = loop
        acc = ((accs[0] + accs[1]) + (accs[2] + accs[3])) + (
            (accs[4] + accs[5]) + (accs[6] + accs[7]))
        off = g * L
        e = energies_v[pl.ds(off, L)]
        out_v[pl.ds(off, L)] = e - acc
        return 0

    lax.fori_loop(0, CGROUPS, group_fn, 0)
    pltpu.sync_copy(out_v, out_hbm.at[pl.ds(base, MPW)])


def _sc_part(energies, species_t, table4):
    mesh = plsc.VectorSubcoreMesh(
        core_axis_name="c", subcore_axis_name="s",
        num_cores=NC, num_subcores=NS,
    )
    f = functools.partial(
        pl.kernel,
        mesh=mesh,
        compiler_params=pltpu.CompilerParams(needs_layout_passes=False),
        out_type=jax.ShapeDtypeStruct((SC_B,), jnp.float32),
        scratch_types=[
            pltpu.VMEM((A, MPW), jnp.int32),
            pltpu.VMEM((MPW,), jnp.float32),
            pltpu.VMEM((MPW,), jnp.float32),
            pltpu.VMEM((L,), jnp.float32),
        ],
    )(_sc_body)
    return f(energies, species_t, table4)


def _tc_body(table_ref, energies_ref, species_t_ref, out_ref):
    t0 = table_ref[0]
    d1 = table_ref[1] - t0
    d2 = table_ref[2] - t0
    d3 = table_ref[3] - t0
    s = species_t_ref[...]
    val = jnp.where(s == 1, d1, 0.0)
    val = val + jnp.where(s == 2, d2, 0.0)
    val = val + jnp.where(s == 3, d3, 0.0)
    sae = jnp.sum(val, axis=0) + jnp.float32(A) * t0
    out_ref[...] = energies_ref[...] - sae


def _tc_part(energies, species_t, table4):
    grid = (TC_B // TC_BLK,)
    off = SC_B // TC_BLK
    return pl.pallas_call(
        _tc_body,
        grid_spec=pltpu.PrefetchScalarGridSpec(
            num_scalar_prefetch=1,
            grid=grid,
            in_specs=[
                pl.BlockSpec((TC_BLK,), lambda i, t: (i + off,)),
                pl.BlockSpec((A, TC_BLK), lambda i, t: (0, i + off)),
            ],
            out_specs=pl.BlockSpec((TC_BLK,), lambda i, t: (i,)),
        ),
        out_shape=jax.ShapeDtypeStruct((TC_B,), jnp.float32),
    )(table4, energies, species_t)


@jax.jit
def _sae_kernel(energies, species, table4):
    species_t = lax.transpose(species, (1, 0))
    sc_out = _sc_part(energies, species_t, table4)
    tc_out = _tc_part(energies, species_t, table4)
    return jnp.concatenate([sc_out, tc_out])


def kernel(energies, species, self_energies):
    return _sae_kernel(energies, species.astype(jnp.int32),
                       self_energies.astype(jnp.float32))
